# baseline (speedup 1.0000x reference)
import functools

import jax
import jax.numpy as jnp
from jax import lax
from jax.experimental import pallas as pl
from jax.experimental.pallas import tpu as pltpu
from jax.experimental.pallas import tpu_sc as plsc

_GAMMA = 12.0
_D = 64
_B = 16384
_NC = 2
_NS = 16
_NW = _NC * _NS
_BPW = _B // _NW
_GROUPS = _BPW // 16


def _tec_body(head_idx_hbm, tail_idx_hbm, rel_idx_hbm, ent_hbm, rel_hbm,
              out_hbm, idxh_v, idxt_v, idxr_v, h_v, t_v, r_v, out_v, sem):
    c = lax.axis_index("c")
    s = lax.axis_index("s")
    wid = s * _NC + c
    base = wid * _BPW

    pltpu.sync_copy(head_idx_hbm.at[pl.ds(base, _BPW)], idxh_v)
    pltpu.sync_copy(tail_idx_hbm.at[pl.ds(base, _BPW)], idxt_v)
    pltpu.sync_copy(rel_idx_hbm.at[pl.ds(base, _BPW)], idxr_v)

    cp_h = pltpu.async_copy(ent_hbm.at[idxh_v], h_v, sem)
    cp_t = pltpu.async_copy(ent_hbm.at[idxt_v], t_v, sem)
    cp_r = pltpu.async_copy(rel_hbm.at[idxr_v], r_v, sem)
    cp_h.wait()
    cp_t.wait()
    cp_r.wait()

    lane = lax.iota(jnp.int32, 16)

    def group(g, carry):
        rowbase = g * 16
        vec = jnp.zeros((16,), jnp.float32)
        for i in range(16):
            r = rowbase + i
            acc = jnp.zeros((16,), jnp.float32)
            for cchunk in range(_D // 16):
                hv = h_v[r, pl.ds(cchunk * 16, 16)]
                tv = t_v[r, pl.ds(cchunk * 16, 16)]
                rv = r_v[r, pl.ds(cchunk * 16, 16)]
                acc = acc + jnp.abs(hv + rv - tv)
            vec = jnp.where(lane == i, jnp.sum(acc), vec)
        out_v[pl.ds(rowbase, 16)] = _GAMMA - vec
        return carry

    lax.fori_loop(0, _GROUPS, group, 0)

    pltpu.sync_copy(out_v, out_hbm.at[pl.ds(base, _BPW)])


@functools.partial(
    pl.kernel,
    out_type=jax.ShapeDtypeStruct((_B,), jnp.float32),
    mesh=plsc.VectorSubcoreMesh(core_axis_name="c", subcore_axis_name="s"),
    compiler_params=pltpu.CompilerParams(
        needs_layout_passes=False, use_tc_tiling_on_sc=False),
    scratch_types=[
        pltpu.VMEM((_BPW,), jnp.int32),
        pltpu.VMEM((_BPW,), jnp.int32),
        pltpu.VMEM((_BPW,), jnp.int32),
        pltpu.VMEM((_BPW, _D), jnp.float32),
        pltpu.VMEM((_BPW, _D), jnp.float32),
        pltpu.VMEM((_BPW, _D), jnp.float32),
        pltpu.VMEM((_BPW,), jnp.float32),
        pltpu.SemaphoreType.DMA,
    ],
)
def _transe_sc(*refs):
    _tec_body(*refs)


def kernel(indices, relations, entity_embedding, relation_embedding):
    head_idx = indices[:, 0].astype(jnp.int32)
    tail_idx = indices[:, 1].astype(jnp.int32)
    rel_idx = relations.astype(jnp.int32)
    return _transe_sc(head_idx, tail_idx, rel_idx,
                      entity_embedding, relation_embedding)
